# baseline (speedup 1.0000x reference)
import jax
import jax.numpy as jnp
from jax.experimental import pallas as pl
from jax.experimental.pallas import tpu as pltpu

_B = 8000
_BR = _B // 8
_NB = 125
_Q = 64
_N = 8 * _Q
_INF = float("inf")
_IMAX = 2**31 - 1


def _lex_cmpswap(va, ia, vb, ib):
    take_a = (va < vb) | ((va == vb) & (ia < ib))
    lo_v = jnp.where(take_a, va, vb)
    lo_i = jnp.where(take_a, ia, ib)
    hi_v = jnp.where(take_a, vb, va)
    hi_i = jnp.where(take_a, ib, ia)
    return lo_v, lo_i, hi_v, hi_i


def _merge4(rv, ri, bv, bi):
    l = []
    for i in range(4):
        lo_v, lo_i, _, _ = _lex_cmpswap(
            rv[i:i + 1], ri[i:i + 1], bv[3 - i:4 - i], bi[3 - i:4 - i])
        l.append((lo_v, lo_i))
    v0, i0, v2, i2 = _lex_cmpswap(l[0][0], l[0][1], l[2][0], l[2][1])
    v1, i1, v3, i3 = _lex_cmpswap(l[1][0], l[1][1], l[3][0], l[3][1])
    v0, i0, v1, i1 = _lex_cmpswap(v0, i0, v1, i1)
    v2, i2, v3, i3 = _lex_cmpswap(v2, i2, v3, i3)
    return (jnp.concatenate([v0, v1, v2, v3], axis=0),
            jnp.concatenate([i0, i1, i2, i3], axis=0))


def _roll(x, shift):
    return pltpu.roll(x, shift, 1)


def _lex_fold(mv, mi):
    for sh in (256, 128, 64):
        rv = _roll(mv, sh)
        ri = _roll(mi, sh)
        take = (rv < mv) | ((rv == mv) & (ri < mi))
        mv = jnp.where(take, rv, mv)
        mi = jnp.where(take, ri, mi)
    return mv, mi


def _val_fold(mv):
    for sh in (256, 128, 64):
        mv = jnp.minimum(mv, _roll(mv, sh))
    return mv


def _body(w1_ref, g_ref, qsqt_ref, jvec_ref, kb_ref, ksqb_ref,
          vals_ref, idx_ref, rs_ref, ri_ref, s_ref, mrow_ref, ev_ref, ei_ref):
    b = pl.program_id(0)

    @pl.when(b == 0)
    def _init():
        rs_ref[...] = jnp.full((4, _Q), _INF, jnp.float32)
        ri_ref[...] = jnp.full((4, _Q), _IMAX, jnp.int32)

    kb = kb_ref[...]
    m2qk = jax.lax.dot_general(
        kb, w1_ref[...], (((1,), (0,)), ((), ())),
        preferred_element_type=jnp.float32)
    ksqb = jax.lax.dot_general(
        ksqb_ref[...], g_ref[...], (((1,), (0,)), ((), ())),
        preferred_element_type=jnp.float32,
        precision=jax.lax.Precision.HIGHEST)
    s = (qsqt_ref[...] + m2qk) + ksqb
    ma = jnp.min(s, axis=0, keepdims=True)
    m0 = _val_fold(ma)

    @pl.when(jnp.any(m0[:, 0:_Q] < rs_ref[3:4, :]))
    def _extract():
        s_ref[...] = s
        mrow_ref[...] = m0[:, 0:_Q]
        ev_ref[...] = jnp.full((4, _Q), _INF, jnp.float32)
        ei_ref[...] = jnp.full((4, _Q), _IMAX, jnp.int32)
        jvec = jvec_ref[...]
        rowid = jax.lax.broadcasted_iota(jnp.int32, (_BR, _N), 0)
        keyid = rowid * 8 + jvec
        base = b * _B
        for r in range(4):
            @pl.when(jnp.any(mrow_ref[...] < rs_ref[3:4, :]))
            def _round(r=r):
                sc = s_ref[...]
                ma_r = jnp.min(sc, axis=0, keepdims=True)
                cand = jnp.where(sc == ma_r, rowid, _IMAX)
                amr = jnp.min(cand, axis=0, keepdims=True)
                ki = amr * 8 + jvec
                mv, mi = _lex_fold(ma_r, ki)
                ev_ref[r:r + 1, :] = mv[:, 0:_Q]
                ei_ref[r:r + 1, :] = mi[:, 0:_Q] + base
                if r < 3:
                    masked = jnp.where(keyid == mi, _INF, sc)
                    s_ref[...] = masked
                    nm = _val_fold(jnp.min(masked, axis=0, keepdims=True))
                    mrow_ref[...] = nm[:, 0:_Q]
        nv, ni = _merge4(rs_ref[...], ri_ref[...], ev_ref[...], ei_ref[...])
        rs_ref[...] = nv
        ri_ref[...] = ni

    @pl.when(b == _NB - 1)
    def _fin():
        vals_ref[...] = jnp.sqrt(jnp.maximum(rs_ref[...], 1e-12))
        idx_ref[...] = ri_ref[...]


def kernel(queries, keys, k):
    q = queries
    qsq = jnp.sum(q * q, axis=1)
    qsqt = jnp.tile(qsq[None, :], (1, 8))
    ksq = jnp.sum(keys * keys, axis=1)
    ksqr = ksq.reshape(-1, 8)
    eye8 = jnp.eye(8, dtype=jnp.float32)
    w1 = (eye8[:, None, :, None] *
          (-2.0 * q.T)[None, :, None, :]).reshape(128, _N)
    g = jnp.repeat(eye8, _Q, axis=1)
    jvec = (jnp.arange(_N, dtype=jnp.int32) // _Q)[None, :]
    kr = keys.reshape(-1, 128)
    vals, idx = pl.pallas_call(
        _body,
        grid=(_NB,),
        in_specs=[
            pl.BlockSpec((128, _N), lambda b: (0, 0)),
            pl.BlockSpec((8, _N), lambda b: (0, 0)),
            pl.BlockSpec((1, _N), lambda b: (0, 0)),
            pl.BlockSpec((1, _N), lambda b: (0, 0)),
            pl.BlockSpec((_BR, 128), lambda b: (b, 0)),
            pl.BlockSpec((_BR, 8), lambda b: (b, 0)),
        ],
        out_specs=[
            pl.BlockSpec((4, _Q), lambda b: (0, 0)),
            pl.BlockSpec((4, _Q), lambda b: (0, 0)),
        ],
        out_shape=[
            jax.ShapeDtypeStruct((4, _Q), jnp.float32),
            jax.ShapeDtypeStruct((4, _Q), jnp.int32),
        ],
        scratch_shapes=[
            pltpu.VMEM((4, _Q), jnp.float32),
            pltpu.VMEM((4, _Q), jnp.int32),
            pltpu.VMEM((_BR, _N), jnp.float32),
            pltpu.VMEM((1, _Q), jnp.float32),
            pltpu.VMEM((4, _Q), jnp.float32),
            pltpu.VMEM((4, _Q), jnp.int32),
        ],
    )(w1, g, qsqt, jvec, kr, ksqr)
    top_dist = vals.T
    indices = idx.T + (jnp.asarray(k, jnp.int32) - 4)
    return top_dist, indices


# speedup vs baseline: 1.1297x; 1.1297x over previous
import jax
import jax.numpy as jnp
from jax.experimental import pallas as pl
from jax.experimental.pallas import tpu as pltpu

_B = 8000
_NB = 125
_Q = 64
_S = 8
_BS = _B // _S
_INF = float("inf")
_IMAX = 2**31 - 1


def _lex_cmpswap(va, ia, vb, ib):
    take_a = (va < vb) | ((va == vb) & (ia < ib))
    lo_v = jnp.where(take_a, va, vb)
    lo_i = jnp.where(take_a, ia, ib)
    hi_v = jnp.where(take_a, vb, va)
    hi_i = jnp.where(take_a, ib, ia)
    return lo_v, lo_i, hi_v, hi_i


def _merge4(rv, ri, bv, bi):
    l = []
    for i in range(4):
        lo_v, lo_i, _, _ = _lex_cmpswap(
            rv[i:i + 1], ri[i:i + 1], bv[3 - i:4 - i], bi[3 - i:4 - i])
        l.append((lo_v, lo_i))
    v0, i0, v2, i2 = _lex_cmpswap(l[0][0], l[0][1], l[2][0], l[2][1])
    v1, i1, v3, i3 = _lex_cmpswap(l[1][0], l[1][1], l[3][0], l[3][1])
    v0, i0, v1, i1 = _lex_cmpswap(v0, i0, v1, i1)
    v2, i2, v3, i3 = _lex_cmpswap(v2, i2, v3, i3)
    return (jnp.concatenate([v0, v1, v2, v3], axis=0),
            jnp.concatenate([i0, i1, i2, i3], axis=0))


def _block_min(s):
    m = s[0:_BS]
    for i in range(1, _S):
        m = jnp.minimum(m, s[i * _BS:(i + 1) * _BS])
    return jnp.min(m, axis=0, keepdims=True)


def _body(m2q_ref, qsq_ref, kb_ref, vals_ref, idx_ref,
          rs_ref, ri_ref, s_ref, mrow_ref, ev_ref, ei_ref):
    b = pl.program_id(0)

    @pl.when(b == 0)
    def _init():
        rs_ref[...] = jnp.full((4, _Q), _INF, jnp.float32)
        ri_ref[...] = jnp.full((4, _Q), _IMAX, jnp.int32)

    kb = kb_ref[...]
    m2qk = jax.lax.dot_general(
        kb, m2q_ref[...], (((1,), (1,)), ((), ())),
        preferred_element_type=jnp.float32)
    ksq = jnp.sum(kb * kb, axis=1, keepdims=True)
    s = (qsq_ref[...] + m2qk) + ksq
    m0 = _block_min(s)

    @pl.when(jnp.any(m0 < rs_ref[3:4, :]))
    def _extract():
        s_ref[...] = s
        mrow_ref[...] = m0
        ev_ref[...] = jnp.full((4, _Q), _INF, jnp.float32)
        ei_ref[...] = jnp.full((4, _Q), _IMAX, jnp.int32)
        rowid = jax.lax.broadcasted_iota(jnp.int32, (_B, _Q), 0)
        base = b * _B
        for r in range(4):
            @pl.when(jnp.any(mrow_ref[...] < rs_ref[3:4, :]))
            def _round(r=r):
                sc = s_ref[...]
                m = mrow_ref[...]
                cand = jnp.where(sc == m, rowid, _IMAX)
                am = jnp.min(cand, axis=0, keepdims=True)
                ev_ref[r:r + 1, :] = m
                ei_ref[r:r + 1, :] = am + base
                if r < 3:
                    masked = jnp.where(rowid == am, _INF, sc)
                    s_ref[...] = masked
                    mrow_ref[...] = _block_min(masked)
        nv, ni = _merge4(rs_ref[...], ri_ref[...], ev_ref[...], ei_ref[...])
        rs_ref[...] = nv
        ri_ref[...] = ni

    @pl.when(b == _NB - 1)
    def _fin():
        vals_ref[...] = jnp.sqrt(jnp.maximum(rs_ref[...], 1e-12))
        idx_ref[...] = ri_ref[...]


def kernel(queries, keys, k):
    m2q = -2.0 * queries
    qsq = jnp.sum(queries * queries, axis=1)[None, :]
    vals, idx = pl.pallas_call(
        _body,
        grid=(_NB,),
        in_specs=[
            pl.BlockSpec((_Q, 16), lambda b: (0, 0)),
            pl.BlockSpec((1, _Q), lambda b: (0, 0)),
            pl.BlockSpec((_B, 16), lambda b: (b, 0)),
        ],
        out_specs=[
            pl.BlockSpec((4, _Q), lambda b: (0, 0)),
            pl.BlockSpec((4, _Q), lambda b: (0, 0)),
        ],
        out_shape=[
            jax.ShapeDtypeStruct((4, _Q), jnp.float32),
            jax.ShapeDtypeStruct((4, _Q), jnp.int32),
        ],
        scratch_shapes=[
            pltpu.VMEM((4, _Q), jnp.float32),
            pltpu.VMEM((4, _Q), jnp.int32),
            pltpu.VMEM((_B, _Q), jnp.float32),
            pltpu.VMEM((1, _Q), jnp.float32),
            pltpu.VMEM((4, _Q), jnp.float32),
            pltpu.VMEM((4, _Q), jnp.int32),
        ],
    )(m2q, qsq, keys)
    top_dist = vals.T
    indices = idx.T + (jnp.asarray(k, jnp.int32) - 4)
    return top_dist, indices


# speedup vs baseline: 1.2319x; 1.0905x over previous
import jax
import jax.numpy as jnp
from jax.experimental import pallas as pl
from jax.experimental.pallas import tpu as pltpu

_B = 8000
_NB = 125
_Q = 64
_S = 8
_BS = _B // _S
_INF = float("inf")
_IMAX = 2**31 - 1


def _lex_cmpswap(va, ia, vb, ib):
    take_a = (va < vb) | ((va == vb) & (ia < ib))
    lo_v = jnp.where(take_a, va, vb)
    lo_i = jnp.where(take_a, ia, ib)
    hi_v = jnp.where(take_a, vb, va)
    hi_i = jnp.where(take_a, ib, ia)
    return lo_v, lo_i, hi_v, hi_i


def _merge4(rv, ri, bv, bi):
    l = []
    for i in range(4):
        lo_v, lo_i, _, _ = _lex_cmpswap(
            rv[i:i + 1], ri[i:i + 1], bv[3 - i:4 - i], bi[3 - i:4 - i])
        l.append((lo_v, lo_i))
    v0, i0, v2, i2 = _lex_cmpswap(l[0][0], l[0][1], l[2][0], l[2][1])
    v1, i1, v3, i3 = _lex_cmpswap(l[1][0], l[1][1], l[3][0], l[3][1])
    v0, i0, v1, i1 = _lex_cmpswap(v0, i0, v1, i1)
    v2, i2, v3, i3 = _lex_cmpswap(v2, i2, v3, i3)
    return (jnp.concatenate([v0, v1, v2, v3], axis=0),
            jnp.concatenate([i0, i1, i2, i3], axis=0))


def _block_min(s):
    m = s[0:_BS]
    for i in range(1, _S):
        m = jnp.minimum(m, s[i * _BS:(i + 1) * _BS])
    return jnp.min(m, axis=0, keepdims=True)


def _argmin_rows(s, m, rowid):
    cand = jnp.where(s == m, rowid, _IMAX)
    return jnp.min(cand, axis=0, keepdims=True)


def _body(m2q_ref, qsq_ref, kb_ref, vals_ref, idx_ref,
          rs_ref, ri_ref, s_ref, ap_ref, ev_ref, ei_ref):
    b = pl.program_id(0)

    @pl.when(b == 0)
    def _init():
        rs_ref[...] = jnp.full((4, _Q), _INF, jnp.float32)
        ri_ref[...] = jnp.full((4, _Q), _IMAX, jnp.int32)

    kb = kb_ref[...]
    m2qk = jax.lax.dot_general(
        kb, m2q_ref[...], (((1,), (1,)), ((), ())),
        preferred_element_type=jnp.float32)
    ksq = jnp.sum(kb * kb, axis=1, keepdims=True)
    s = (qsq_ref[...] + m2qk) + ksq
    m0 = _block_min(s)
    t = rs_ref[3:4, :]
    rowid = jax.lax.broadcasted_iota(jnp.int32, (_B, _Q), 0)
    base = b * _B

    @pl.when(jnp.any(m0 < t))
    def _extract():
        cnt = jnp.sum((s < t).astype(jnp.int32), axis=0, keepdims=True)
        maxc = jnp.max(cnt)
        ev_ref[...] = jnp.full((4, _Q), _INF, jnp.float32)
        ei_ref[...] = jnp.full((4, _Q), _IMAX, jnp.int32)
        am0 = _argmin_rows(s, m0, rowid)
        ev_ref[0:1, :] = m0
        ei_ref[0:1, :] = am0 + base
        ap_ref[...] = am0

        @pl.when(maxc > 1)
        def _more_rounds():
            s_ref[...] = jnp.where(rowid == ap_ref[...], _INF, s)
            for r in range(1, 4):
                @pl.when(maxc > r)
                def _round(r=r):
                    sc = s_ref[...]
                    m = _block_min(sc)
                    am = _argmin_rows(sc, m, rowid)
                    ev_ref[r:r + 1, :] = m
                    ei_ref[r:r + 1, :] = am + base
                    if r < 3:
                        s_ref[...] = jnp.where(rowid == am, _INF, sc)

        nv, ni = _merge4(rs_ref[...], ri_ref[...], ev_ref[...], ei_ref[...])
        rs_ref[...] = nv
        ri_ref[...] = ni

    @pl.when(b == _NB - 1)
    def _fin():
        vals_ref[...] = jnp.sqrt(jnp.maximum(rs_ref[...], 1e-12))
        idx_ref[...] = ri_ref[...]


def kernel(queries, keys, k):
    m2q = -2.0 * queries
    qsq = jnp.sum(queries * queries, axis=1)[None, :]
    vals, idx = pl.pallas_call(
        _body,
        grid=(_NB,),
        in_specs=[
            pl.BlockSpec((_Q, 16), lambda b: (0, 0)),
            pl.BlockSpec((1, _Q), lambda b: (0, 0)),
            pl.BlockSpec((_B, 16), lambda b: (b, 0)),
        ],
        out_specs=[
            pl.BlockSpec((4, _Q), lambda b: (0, 0)),
            pl.BlockSpec((4, _Q), lambda b: (0, 0)),
        ],
        out_shape=[
            jax.ShapeDtypeStruct((4, _Q), jnp.float32),
            jax.ShapeDtypeStruct((4, _Q), jnp.int32),
        ],
        scratch_shapes=[
            pltpu.VMEM((4, _Q), jnp.float32),
            pltpu.VMEM((4, _Q), jnp.int32),
            pltpu.VMEM((_B, _Q), jnp.float32),
            pltpu.VMEM((1, _Q), jnp.int32),
            pltpu.VMEM((4, _Q), jnp.float32),
            pltpu.VMEM((4, _Q), jnp.int32),
        ],
    )(m2q, qsq, keys)
    top_dist = vals.T
    indices = idx.T + (jnp.asarray(k, jnp.int32) - 4)
    return top_dist, indices
